# Initial kernel scaffold; baseline (speedup 1.0000x reference)
import jax
import jax.numpy as jnp
from jax.experimental import pallas as pl


def kernel(x, table, W):
    raise NotImplementedError("write your pallas kernel here")


# speedup vs baseline: 15.0571x; 15.0571x over previous
import functools

import jax
import jax.numpy as jnp
from jax import lax
from jax.experimental import pallas as pl
from jax.experimental.pallas import tpu as pltpu
from jax.experimental.pallas import tpu_sc as plsc

FACTOR = 16
EMBED = 64
NC = 2
NS = 16
NW = NC * NS

KJ = 8


def _make_gather(n_rows):
    rows_per_w = n_rows // NW
    n_outer = rows_per_w // KJ
    mesh = plsc.VectorSubcoreMesh(core_axis_name="c", subcore_axis_name="s")

    @functools.partial(
        pl.kernel,
        mesh=mesh,
        out_type=jax.ShapeDtypeStruct((n_rows, 128, FACTOR), jnp.float32),
        scratch_types=[
            pltpu.VMEM((KJ, 128), jnp.int32),
            pltpu.VMEM((KJ, 128, FACTOR), jnp.float32),
            pltpu.SemaphoreType.DMA,
        ],
        compiler_params=pltpu.CompilerParams(use_tc_tiling_on_sc=False),
    )
    def gather(table_h, idx_h, out_h, idx_v, rows_v, sem):
        wid = lax.axis_index("s") * NC + lax.axis_index("c")
        base = wid * rows_per_w

        def outer(j, carry):
            r0 = base + j * KJ
            pltpu.sync_copy(idx_h.at[pl.ds(r0, KJ)], idx_v)
            copies = [
                pltpu.async_copy(table_h.at[idx_v.at[jj]], rows_v.at[jj], sem)
                for jj in range(KJ)
            ]
            for cp in copies:
                cp.wait()
            pltpu.sync_copy(rows_v, out_h.at[pl.ds(r0, KJ)])
            return carry

        lax.fori_loop(0, n_outer, outer, 0)

    return gather


def _proj_body(f_ref, g_ref, o_ref):
    o_ref[...] = jnp.dot(f_ref[...], g_ref[...],
                         preferred_element_type=jnp.float32)


def _project(f2, g):
    n2 = f2.shape[0]
    rb = 1024
    return pl.pallas_call(
        _proj_body,
        grid=(n2 // rb,),
        in_specs=[
            pl.BlockSpec((rb, 128), lambda i: (i, 0)),
            pl.BlockSpec((128, 8 * EMBED), lambda i: (0, 0)),
        ],
        out_specs=pl.BlockSpec((rb, 8 * EMBED), lambda i: (i, 0)),
        out_shape=jax.ShapeDtypeStruct((n2, 8 * EMBED), jnp.float32),
    )(f2, g)


def kernel(x, table, W):
    b, l = x.shape
    n = b * l
    idx2 = x.reshape(n // 128, 128).astype(jnp.int32)
    fact = _make_gather(n // 128)(table, idx2)
    f2 = fact.reshape(n // 8, 8 * FACTOR)
    g = jnp.kron(jnp.eye(8, dtype=jnp.float32), W.T)
    o2 = _project(f2, g)
    return o2.reshape(b, l, EMBED)


# speedup vs baseline: 18.1273x; 1.2039x over previous
import functools

import jax
import jax.numpy as jnp
from jax import lax
from jax.experimental import pallas as pl
from jax.experimental.pallas import tpu as pltpu
from jax.experimental.pallas import tpu_sc as plsc

FACTOR = 16
EMBED = 64
NC = 2
NS = 16
NW = NC * NS

KJ = 8
CHUNK = KJ * 128
SKEW = CHUNK + 1


def _make_gather(n_l, n_b):
    n_chunks = (n_l * n_b) // CHUNK
    chunks_per_w = n_chunks // NW
    cb_per_l = n_b // CHUNK
    mesh = plsc.VectorSubcoreMesh(core_axis_name="c", subcore_axis_name="s")

    @functools.partial(
        pl.kernel,
        mesh=mesh,
        out_type=jax.ShapeDtypeStruct((n_l * FACTOR, n_b), jnp.float32),
        scratch_types=[
            pltpu.VMEM((KJ, 128), jnp.int32),
            pltpu.VMEM((CHUNK, FACTOR), jnp.float32),
            pltpu.VMEM((FACTOR, SKEW), jnp.float32),
            pltpu.SemaphoreType.DMA,
        ],
        compiler_params=pltpu.CompilerParams(
            use_tc_tiling_on_sc=False, needs_layout_passes=False),
    )
    def gather(table_h, idx_h, out_h, idx_v, rows_v, trans_v, sem):
        wid = lax.axis_index("s") * NC + lax.axis_index("c")
        base = wid * chunks_per_w
        fidx = lax.iota(jnp.int32, 16)
        zeros = jnp.zeros((16,), jnp.int32)

        def chunk_body(ci, carry):
            c = base + ci
            l = c // cb_per_l
            cb = c % cb_per_l
            pltpu.sync_copy(idx_h.at[pl.ds(c * KJ, KJ)], idx_v)
            copies = [
                pltpu.async_copy(
                    table_h.at[idx_v.at[jj]],
                    rows_v.at[pl.ds(jj * 128, 128)],
                    sem,
                )
                for jj in range(KJ)
            ]
            for cp in copies:
                cp.wait()

            def t_body(t, carry2):
                for jj in range(16):
                    j = t * 16 + jj
                    val = rows_v[j, :]
                    plsc.store_scatter(trans_v, [fidx, zeros + j], val)
                return carry2

            lax.fori_loop(0, CHUNK // 16, t_body, 0)

            pltpu.sync_copy(
                trans_v.at[:, pl.ds(0, CHUNK)],
                out_h.at[pl.ds(l * FACTOR, FACTOR), pl.ds(cb * CHUNK, CHUNK)],
            )
            return carry

        lax.fori_loop(0, chunks_per_w, chunk_body, 0)

    return gather


C = 2048


def _proj_body(ft_ref, w_ref, o_ref):
    o_ref[0] = lax.dot_general(
        w_ref[...], ft_ref[...], (((1,), (0,)), ((), ())),
        preferred_element_type=jnp.float32)


def _project(ft, w, n_l, n_b):
    return pl.pallas_call(
        _proj_body,
        grid=(n_l, n_b // C),
        in_specs=[
            pl.BlockSpec((FACTOR, C), lambda l, c: (l, c)),
            pl.BlockSpec((EMBED, FACTOR), lambda l, c: (0, 0)),
        ],
        out_specs=pl.BlockSpec((1, EMBED, C), lambda l, c: (l, 0, c)),
        out_shape=jax.ShapeDtypeStruct((n_l, EMBED, n_b), jnp.float32),
    )(ft, w)


def kernel(x, table, W):
    b, l = x.shape
    n = b * l
    idx2 = x.T.reshape(n // 128, 128).astype(jnp.int32)
    ft = _make_gather(l, b)(table, idx2)
    o3 = _project(ft, W, l, b)
    return jnp.transpose(o3, (2, 0, 1))


# speedup vs baseline: 22.5498x; 1.2440x over previous
import functools

import jax
import jax.numpy as jnp
from jax import lax
from jax.experimental import pallas as pl
from jax.experimental.pallas import tpu as pltpu
from jax.experimental.pallas import tpu_sc as plsc

FACTOR = 16
EMBED = 64
NC = 2
NS = 16
NW = NC * NS

KJ = 8
CHUNK = KJ * 128
SKEW = CHUNK + 1


def _make_gather(n_l, n_b):
    n_chunks = (n_l * n_b) // CHUNK
    chunks_per_w = n_chunks // NW
    cb_per_l = n_b // CHUNK
    mesh = plsc.VectorSubcoreMesh(core_axis_name="c", subcore_axis_name="s")

    @functools.partial(
        pl.kernel,
        mesh=mesh,
        out_type=jax.ShapeDtypeStruct((n_l * FACTOR, n_b), jnp.float32),
        scratch_types=[
            pltpu.VMEM((KJ, 128), jnp.int32),
            pltpu.VMEM((CHUNK, FACTOR), jnp.float32),
            pltpu.VMEM((FACTOR, SKEW), jnp.float32),
            pltpu.SemaphoreType.DMA,
        ],
        compiler_params=pltpu.CompilerParams(
            use_tc_tiling_on_sc=False, needs_layout_passes=False),
    )
    def gather(table_h, idx_h, out_h, idx_v, rows_v, trans_v, sem):
        wid = lax.axis_index("s") * NC + lax.axis_index("c")
        base = wid * chunks_per_w
        fidx = lax.iota(jnp.int32, 16)
        zeros = jnp.zeros((16,), jnp.int32)

        def chunk_body(ci, carry):
            c = base + ci
            l = c // cb_per_l
            cb = c % cb_per_l
            pltpu.sync_copy(idx_h.at[pl.ds(c * KJ, KJ)], idx_v)
            copies = [
                pltpu.async_copy(
                    table_h.at[idx_v.at[jj]],
                    rows_v.at[pl.ds(jj * 128, 128)],
                    sem,
                )
                for jj in range(KJ)
            ]
            for cp in copies:
                cp.wait()

            def t_body(t, carry2):
                for jj in range(16):
                    j = t * 16 + jj
                    val = rows_v[j, :]
                    plsc.store_scatter(trans_v, [fidx, zeros + j], val)
                return carry2

            lax.fori_loop(0, CHUNK // 16, t_body, 0)

            pltpu.sync_copy(
                trans_v.at[:, pl.ds(0, CHUNK)],
                out_h.at[pl.ds(l * FACTOR, FACTOR), pl.ds(cb * CHUNK, CHUNK)],
            )
            return carry

        lax.fori_loop(0, chunks_per_w, chunk_body, 0)

    return gather


C = 16384


def _proj_body(ft_ref, w_ref, o_ref):
    o_ref[0] = lax.dot_general(
        w_ref[...], ft_ref[...], (((1,), (0,)), ((), ())),
        preferred_element_type=jnp.float32)


def _project(ft, w, n_l, n_b):
    return pl.pallas_call(
        _proj_body,
        grid=(n_l, n_b // C),
        in_specs=[
            pl.BlockSpec((FACTOR, C), lambda l, c: (l, c)),
            pl.BlockSpec((EMBED, FACTOR), lambda l, c: (0, 0)),
        ],
        out_specs=pl.BlockSpec((1, EMBED, C), lambda l, c: (l, 0, c)),
        out_shape=jax.ShapeDtypeStruct((n_l, EMBED, n_b), jnp.float32),
    )(ft, w)


def kernel(x, table, W):
    b, l = x.shape
    n = b * l
    idx2 = x.T.reshape(n // 128, 128).astype(jnp.int32)
    ft = _make_gather(l, b)(table, idx2)
    o3 = _project(ft, W, l, b)
    return jnp.transpose(o3, (2, 0, 1))
